# baseline (speedup 1.0000x reference)
import functools

import jax
import jax.numpy as jnp
from jax import lax
from jax.experimental import pallas as pl
from jax.experimental.pallas import tpu as pltpu
from jax.experimental.pallas import tpu_sc as plsc

B = 4096
NF = 26
SV = 100001
SD = 16
IV = 1000001
ID = 64
L = 200
DNN = 256
H = 128

NC = 2
NS = 16
LANES = 16
NW = NC * NS
BPW = B // NW
FPW = BPW * NF * SD

_mesh = plsc.VectorSubcoreMesh(
    core_axis_name="c", subcore_axis_name="s", num_cores=NC, num_subcores=NS
)

IVP = 62 * 16384
IVR = 1000008
_VC = 16384


def _pack_body(it_ref, out_ref):
    out_ref[...] = jnp.pad(it_ref[...].T.astype(jnp.bfloat16),
                           ((0, 0), (0, ID)))


def _item_pack(item_t):
    return pl.pallas_call(
        _pack_body,
        grid=(IVP // _VC,),
        in_specs=[pl.BlockSpec((ID, _VC), lambda i: (0, i))],
        out_specs=pl.BlockSpec((_VC, 2 * ID), lambda i: (i, 0)),
        out_shape=jax.ShapeDtypeStruct((IVR, 2 * ID), jnp.bfloat16),
    )(item_t)


@functools.partial(
    pl.kernel,
    out_type=(
        jax.ShapeDtypeStruct((B * NF * SD,), jnp.float32),
        jax.ShapeDtypeStruct((B, ID), jnp.float32),
    ),
    mesh=_mesh,
    scratch_types=[
        pltpu.VMEM((NF, BPW), jnp.int32),
        pltpu.VMEM((FPW // 4,), jnp.int32),
        pltpu.VMEM((FPW // 4,), jnp.float32),
        pltpu.VMEM((BPW * L,), jnp.int32),
        pltpu.VMEM((L, 2 * ID), jnp.bfloat16),
        pltpu.VMEM((L, 2 * ID), jnp.bfloat16),
        pltpu.VMEM((BPW, ID), jnp.float32),
        pltpu.SemaphoreType.DMA,
        pltpu.SemaphoreType.DMA,
        pltpu.SemaphoreType.DMA,
    ],
    compiler_params=pltpu.CompilerParams(use_tc_tiling_on_sc=False,
                                         needs_layout_passes=False),
)
def _sc_gather(sf_hbm, hist_hbm, tables_hbm, item_hbm,
               flat_out, pooled_out,
               sfi_v, addr_v, svals_v, hidx_v, hbuf0, hbuf1, acc_v,
               sem_s, sem_h0, sem_h1):
    wid = lax.axis_index("s") * NC + lax.axis_index("c")

    for f in range(NF):
        pltpu.async_copy(sf_hbm.at[f, pl.ds(wid * BPW, BPW)], sfi_v.at[f],
                         sem_s)
    pltpu.sync_copy(hist_hbm.at[pl.ds(wid * (BPW * L), BPW * L)], hidx_v)

    def _hist_copies(e, buf, sem):
        d1 = pltpu.make_async_copy(
            item_hbm.at[hidx_v.at[pl.ds(e * L, 128)]],
            buf.at[pl.ds(0, 128)], sem)
        d2 = pltpu.make_async_copy(
            item_hbm.at[hidx_v.at[pl.ds(e * L + 128, L - 128)]],
            buf.at[pl.ds(128, L - 128)], sem)
        return d1, d2

    def _hstart(e, buf, sem):
        for d in _hist_copies(e, buf, sem):
            d.start()

    def _hwait(e, buf, sem):
        for d in _hist_copies(e, buf, sem):
            d.wait()

    _hstart(0, hbuf0, sem_h0)
    _hstart(1, hbuf1, sem_h1)

    iota = lax.broadcasted_iota(jnp.int32, (LANES,), 0)
    for f in range(NF):
        pltpu.make_async_copy(sf_hbm.at[f, pl.ds(wid * BPW, BPW)],
                              sfi_v.at[f], sem_s).wait()

    def _build_addrs(quarter):
        b0 = quarter * (BPW // 4)

        def _abody(i, _):
            f = i // (BPW // (4 * LANES))
            c = i % (BPW // (4 * LANES))
            v = sfi_v[f, pl.ds(b0 + c * LANES, LANES)]
            w = lax.shift_right_logical(v, 7) * (8 * 128) + (v & 127)
            dst0 = (c * LANES + iota) * (NF * SD) + f * SD
            for d in range(SD):
                src = w + (f * 2 + d // 8) * (782 * 8 * 128) + (d % 8) * 128
                plsc.store_scatter(addr_v, [dst0 + d], src)
            return 0

        lax.fori_loop(0, NF * (BPW // (4 * LANES)), _abody, 0)

    _sgat = lambda: pltpu.make_async_copy(tables_hbm.at[addr_v], svals_v,
                                          sem_s)
    _build_addrs(0)
    _sgat().start()

    def _accum_store(buf, e):
        zero = jnp.zeros((LANES,), jnp.float32)

        def body(i, accs):
            r = i * 8
            a0, a1, a2, a3 = accs
            for rr in range(8):
                e0, o0 = plsc.unpack(buf[r + rr, pl.ds(0, 2 * LANES)],
                                     format=plsc.PackFormat.INTERLEAVED)
                e1, o1 = plsc.unpack(buf[r + rr, pl.ds(2 * LANES, 2 * LANES)],
                                     format=plsc.PackFormat.INTERLEAVED)
                a0 = a0 + e0
                a1 = a1 + o0
                a2 = a2 + e1
                a3 = a3 + o1
            return a0, a1, a2, a3

        a0, a1, a2, a3 = lax.fori_loop(0, L // 8, body, (zero,) * 4)
        acc_v[e, pl.ds(0, LANES)] = a0
        acc_v[e, pl.ds(LANES, LANES)] = a1
        acc_v[e, pl.ds(2 * LANES, LANES)] = a2
        acc_v[e, pl.ds(3 * LANES, LANES)] = a3

    def _outer(g, _):
        e0 = 2 * g
        _hwait(e0, hbuf0, sem_h0)
        _hstart(e0 + 2, hbuf0, sem_h0)
        _accum_store(hbuf0, e0)
        _hwait(e0 + 1, hbuf1, sem_h1)
        _hstart(e0 + 3, hbuf1, sem_h1)
        _accum_store(hbuf1, e0 + 1)
        return 0

    FQ = FPW // 4
    lax.fori_loop(0, 16, _outer, 0)
    for q in range(1, 4):
        _sgat().wait()
        pltpu.sync_copy(svals_v,
                        flat_out.at[pl.ds(wid * FPW + (q - 1) * FQ, FQ)])
        _build_addrs(q)
        _sgat().start()
        lax.fori_loop(16 * q, min(16 * (q + 1), BPW // 2 - 1), _outer, 0)
    _hwait(BPW - 2, hbuf0, sem_h0)
    _accum_store(hbuf0, BPW - 2)
    _hwait(BPW - 1, hbuf1, sem_h1)
    _accum_store(hbuf1, BPW - 1)

    pltpu.sync_copy(acc_v, pooled_out.at[pl.ds(wid * BPW, BPW)])

    _sgat().wait()
    pltpu.sync_copy(svals_v, flat_out.at[pl.ds(wid * FPW + 3 * FQ, FQ)])


def _mlp_body(flat_ref, pooled_ref, hist_ref, w1a_ref, w1b_ref, b1_ref,
              w2_ref, b2_ref, out_ref):
    cnt = jnp.sum((hist_ref[...] != 0).astype(jnp.float32), axis=1,
                  keepdims=True)
    p = pooled_ref[...] / jnp.maximum(cnt, 1.0)
    h = jnp.dot(flat_ref[...], w1a_ref[...],
                preferred_element_type=jnp.float32)
    h = h + jnp.dot(p, w1b_ref[...], preferred_element_type=jnp.float32)
    h = jnp.maximum(h + b1_ref[...], 0.0)
    out_ref[...] = jnp.dot(h, w2_ref[...],
                           preferred_element_type=jnp.float32) + b2_ref[...]


def _mlp(flat, pooled, history, w1a, w1b, b1, w2, b2):
    BT = 1024
    return pl.pallas_call(
        _mlp_body,
        grid=(B // BT,),
        in_specs=[
            pl.BlockSpec((BT, NF * SD), lambda i: (i, 0)),
            pl.BlockSpec((BT, ID), lambda i: (i, 0)),
            pl.BlockSpec((BT, L), lambda i: (i, 0)),
            pl.BlockSpec((NF * SD, DNN), lambda i: (0, 0)),
            pl.BlockSpec((ID, DNN), lambda i: (0, 0)),
            pl.BlockSpec((1, DNN), lambda i: (0, 0)),
            pl.BlockSpec((DNN, H), lambda i: (0, 0)),
            pl.BlockSpec((1, H), lambda i: (0, 0)),
        ],
        out_specs=pl.BlockSpec((BT, H), lambda i: (i, 0)),
        out_shape=jax.ShapeDtypeStruct((B, H), jnp.float32),
    )(flat, pooled, history, w1a, w1b, b1, w2, b2)


def kernel(sparse_feats, history, sparse_tables, item_table, W1, b1, W2, b2):
    sf_t = sparse_feats.T
    hist_flat = history.reshape(B * L)
    SVP = 782 * 128
    tables_flat = (
        jnp.pad(sparse_tables, ((0, 0), (0, SVP - SV), (0, 0)))
        .transpose(0, 2, 1)
        .reshape(NF, 2, 8, 782, 128)
        .transpose(0, 1, 3, 2, 4)
        .reshape(NF * SD * SVP))
    flat, pooled = _sc_gather(sf_t, hist_flat, tables_flat,
                              _item_pack(item_table.T))
    half = jnp.arange(LANES)
    perm = jnp.concatenate([2 * half, 2 * half + 1,
                            32 + 2 * half, 33 + 2 * half])
    w1b = W1[NF * SD:][perm]
    return _mlp(flat.reshape(B, NF * SD), pooled, history,
                W1[:NF * SD], w1b, b1.reshape(1, DNN),
                W2, b2.reshape(1, H))


# speedup vs baseline: 2.4735x; 2.4735x over previous
import functools

import jax
import jax.numpy as jnp
from jax import lax
from jax.experimental import pallas as pl
from jax.experimental.pallas import tpu as pltpu
from jax.experimental.pallas import tpu_sc as plsc

B = 4096
NF = 26
SV = 100001
SD = 16
IV = 1000001
ID = 64
L = 200
DNN = 256
H = 128

NC = 2
NS = 16
LANES = 16
NW = NC * NS
BPW = B // NW
FPW = BPW * NF * SD

_mesh = plsc.VectorSubcoreMesh(
    core_axis_name="c", subcore_axis_name="s", num_cores=NC, num_subcores=NS
)

IVP = 31 * 32768
IVR = 1000008
_VC = 32768


def _pack_body(it_ref, out_ref):
    out_ref[...] = jnp.pad(it_ref[...].T, ((0, 0), (0, ID)))


def _item_pack(item_t):
    return pl.pallas_call(
        _pack_body,
        grid=(IVP // _VC,),
        in_specs=[pl.BlockSpec((ID, _VC), lambda i: (0, i))],
        out_specs=pl.BlockSpec((_VC, 2 * ID), lambda i: (i, 0)),
        out_shape=jax.ShapeDtypeStruct((IVR, 2 * ID), jnp.float32),
    )(item_t)


@functools.partial(
    pl.kernel,
    out_type=(
        jax.ShapeDtypeStruct((B * NF * SD,), jnp.float32),
        jax.ShapeDtypeStruct((B, ID), jnp.float32),
    ),
    mesh=_mesh,
    scratch_types=[
        pltpu.VMEM((NF, BPW), jnp.int32),
        pltpu.VMEM((FPW // 4,), jnp.int32),
        pltpu.VMEM((FPW // 4,), jnp.float32),
        pltpu.VMEM((BPW * L,), jnp.int32),
        pltpu.VMEM((L, 2 * ID), jnp.float32),
        pltpu.VMEM((L, 2 * ID), jnp.float32),
        pltpu.VMEM((BPW, ID), jnp.float32),
        pltpu.SemaphoreType.DMA,
        pltpu.SemaphoreType.DMA,
        pltpu.SemaphoreType.DMA,
    ],
    compiler_params=pltpu.CompilerParams(use_tc_tiling_on_sc=False,
                                         needs_layout_passes=False),
)
def _sc_gather(sf_hbm, hist_hbm, tables_hbm, item_hbm,
               flat_out, pooled_out,
               sfi_v, addr_v, svals_v, hidx_v, hbuf0, hbuf1, acc_v,
               sem_s, sem_h0, sem_h1):
    wid = lax.axis_index("s") * NC + lax.axis_index("c")

    for f in range(NF):
        pltpu.async_copy(sf_hbm.at[f, pl.ds(wid * BPW, BPW)], sfi_v.at[f],
                         sem_s)
    pltpu.sync_copy(hist_hbm.at[pl.ds(wid * (BPW * L), BPW * L)], hidx_v)

    def _hist_copies(e, buf, sem):
        d1 = pltpu.make_async_copy(
            item_hbm.at[hidx_v.at[pl.ds(e * L, 128)]],
            buf.at[pl.ds(0, 128)], sem)
        d2 = pltpu.make_async_copy(
            item_hbm.at[hidx_v.at[pl.ds(e * L + 128, L - 128)]],
            buf.at[pl.ds(128, L - 128)], sem)
        return d1, d2

    def _hstart(e, buf, sem):
        for d in _hist_copies(e, buf, sem):
            d.start()

    def _hwait(e, buf, sem):
        for d in _hist_copies(e, buf, sem):
            d.wait()

    _hstart(0, hbuf0, sem_h0)
    _hstart(1, hbuf1, sem_h1)

    iota = lax.broadcasted_iota(jnp.int32, (LANES,), 0)
    for f in range(NF):
        pltpu.make_async_copy(sf_hbm.at[f, pl.ds(wid * BPW, BPW)],
                              sfi_v.at[f], sem_s).wait()

    def _build_addrs(quarter):
        b0 = quarter * (BPW // 4)

        def _abody(i, _):
            f = i // (BPW // (4 * LANES))
            c = i % (BPW // (4 * LANES))
            v = sfi_v[f, pl.ds(b0 + c * LANES, LANES)]
            w = lax.shift_right_logical(v, 7) * (8 * 128) + (v & 127)
            dst0 = (c * LANES + iota) * (NF * SD) + f * SD
            for d in range(SD):
                src = w + (f * 2 + d // 8) * (782 * 8 * 128) + (d % 8) * 128
                plsc.store_scatter(addr_v, [dst0 + d], src)
            return 0

        lax.fori_loop(0, NF * (BPW // (4 * LANES)), _abody, 0)

    _sgat = lambda: pltpu.make_async_copy(tables_hbm.at[addr_v], svals_v,
                                          sem_s)
    _build_addrs(0)
    _sgat().start()

    def _accum_store(buf, e):
        zero = jnp.zeros((LANES,), jnp.float32)

        def body(i, accs):
            r = i * 8
            a0, a1, a2, a3 = accs
            for rr in range(8):
                a0 = a0 + buf[r + rr, pl.ds(0, LANES)]
                a1 = a1 + buf[r + rr, pl.ds(LANES, LANES)]
                a2 = a2 + buf[r + rr, pl.ds(2 * LANES, LANES)]
                a3 = a3 + buf[r + rr, pl.ds(3 * LANES, LANES)]
            return a0, a1, a2, a3

        a0, a1, a2, a3 = lax.fori_loop(0, L // 8, body, (zero,) * 4)
        acc_v[e, pl.ds(0, LANES)] = a0
        acc_v[e, pl.ds(LANES, LANES)] = a1
        acc_v[e, pl.ds(2 * LANES, LANES)] = a2
        acc_v[e, pl.ds(3 * LANES, LANES)] = a3

    def _outer(g, _):
        e0 = 2 * g
        _hwait(e0, hbuf0, sem_h0)
        _hstart(e0 + 2, hbuf0, sem_h0)
        _accum_store(hbuf0, e0)
        _hwait(e0 + 1, hbuf1, sem_h1)
        _hstart(e0 + 3, hbuf1, sem_h1)
        _accum_store(hbuf1, e0 + 1)
        return 0

    FQ = FPW // 4
    lax.fori_loop(0, 16, _outer, 0)
    for q in range(1, 4):
        _sgat().wait()
        pltpu.sync_copy(svals_v,
                        flat_out.at[pl.ds(wid * FPW + (q - 1) * FQ, FQ)])
        _build_addrs(q)
        _sgat().start()
        lax.fori_loop(16 * q, min(16 * (q + 1), BPW // 2 - 1), _outer, 0)
    _hwait(BPW - 2, hbuf0, sem_h0)
    _accum_store(hbuf0, BPW - 2)
    _hwait(BPW - 1, hbuf1, sem_h1)
    _accum_store(hbuf1, BPW - 1)

    pltpu.sync_copy(acc_v, pooled_out.at[pl.ds(wid * BPW, BPW)])

    _sgat().wait()
    pltpu.sync_copy(svals_v, flat_out.at[pl.ds(wid * FPW + 3 * FQ, FQ)])


def _mlp_body(flat_ref, pooled_ref, hist_ref, w1a_ref, w1b_ref, b1_ref,
              w2_ref, b2_ref, out_ref):
    cnt = jnp.sum((hist_ref[...] != 0).astype(jnp.float32), axis=1,
                  keepdims=True)
    p = pooled_ref[...] / jnp.maximum(cnt, 1.0)
    h = jnp.dot(flat_ref[...], w1a_ref[...],
                preferred_element_type=jnp.float32)
    h = h + jnp.dot(p, w1b_ref[...], preferred_element_type=jnp.float32)
    h = jnp.maximum(h + b1_ref[...], 0.0)
    out_ref[...] = jnp.dot(h, w2_ref[...],
                           preferred_element_type=jnp.float32) + b2_ref[...]


def _mlp(flat, pooled, history, w1a, w1b, b1, w2, b2):
    BT = 1024
    return pl.pallas_call(
        _mlp_body,
        grid=(B // BT,),
        in_specs=[
            pl.BlockSpec((BT, NF * SD), lambda i: (i, 0)),
            pl.BlockSpec((BT, ID), lambda i: (i, 0)),
            pl.BlockSpec((BT, L), lambda i: (i, 0)),
            pl.BlockSpec((NF * SD, DNN), lambda i: (0, 0)),
            pl.BlockSpec((ID, DNN), lambda i: (0, 0)),
            pl.BlockSpec((1, DNN), lambda i: (0, 0)),
            pl.BlockSpec((DNN, H), lambda i: (0, 0)),
            pl.BlockSpec((1, H), lambda i: (0, 0)),
        ],
        out_specs=pl.BlockSpec((BT, H), lambda i: (i, 0)),
        out_shape=jax.ShapeDtypeStruct((B, H), jnp.float32),
    )(flat, pooled, history, w1a, w1b, b1, w2, b2)


def kernel(sparse_feats, history, sparse_tables, item_table, W1, b1, W2, b2):
    sf_t = sparse_feats.T
    hist_flat = history.reshape(B * L)
    SVP = 782 * 128
    tables_flat = (
        jnp.pad(sparse_tables, ((0, 0), (0, SVP - SV), (0, 0)))
        .transpose(0, 2, 1)
        .reshape(NF, 2, 8, 782, 128)
        .transpose(0, 1, 3, 2, 4)
        .reshape(NF * SD * SVP))
    flat, pooled = _sc_gather(sf_t, hist_flat, tables_flat,
                              _item_pack(item_table.T))
    return _mlp(flat.reshape(B, NF * SD), pooled, history,
                W1[:NF * SD], W1[NF * SD:], b1.reshape(1, DNN),
                W2, b2.reshape(1, H))


# speedup vs baseline: 2.7139x; 1.0972x over previous
import functools

import jax
import jax.numpy as jnp
from jax import lax
from jax.experimental import pallas as pl
from jax.experimental.pallas import tpu as pltpu
from jax.experimental.pallas import tpu_sc as plsc

B = 4096
NF = 26
SV = 100001
SD = 16
IV = 1000001
ID = 64
L = 200
DNN = 256
H = 128

NC = 2
NS = 16
LANES = 16
NW = NC * NS
BPW = B // NW
FPW = BPW * NF * SD

_mesh = plsc.VectorSubcoreMesh(
    core_axis_name="c", subcore_axis_name="s", num_cores=NC, num_subcores=NS
)

IVP = 62 * 16384
IVR = 1000008
_VC = 16384


def _pack_body(it_ref, out_ref):
    out_ref[...] = jnp.pad(it_ref[...].T, ((0, 0), (0, ID)))


def _item_pack(item_t):
    return pl.pallas_call(
        _pack_body,
        grid=(IVP // _VC,),
        in_specs=[pl.BlockSpec((ID, _VC), lambda i: (0, i))],
        out_specs=pl.BlockSpec((_VC, 2 * ID), lambda i: (i, 0)),
        out_shape=jax.ShapeDtypeStruct((IVR, 2 * ID), jnp.float32),
    )(item_t)


@functools.partial(
    pl.kernel,
    out_type=(
        jax.ShapeDtypeStruct((B * NF * SD,), jnp.float32),
        jax.ShapeDtypeStruct((B, ID), jnp.float32),
    ),
    mesh=_mesh,
    scratch_types=[
        pltpu.VMEM((NF, BPW), jnp.int32),
        pltpu.VMEM((FPW // 4,), jnp.int32),
        pltpu.VMEM((FPW // 4,), jnp.float32),
        pltpu.VMEM((BPW * L,), jnp.int32),
        pltpu.VMEM((L, ID), jnp.float32),
        pltpu.VMEM((L, ID), jnp.float32),
        pltpu.VMEM((BPW, ID), jnp.float32),
        pltpu.SemaphoreType.DMA,
        pltpu.SemaphoreType.DMA,
        pltpu.SemaphoreType.DMA,
    ],
    compiler_params=pltpu.CompilerParams(use_tc_tiling_on_sc=False,
                                         needs_layout_passes=False),
)
def _sc_gather(sf_hbm, hist_hbm, tables_hbm, item_hbm,
               flat_out, pooled_out,
               sfi_v, addr_v, svals_v, hidx_v, hbuf0, hbuf1, acc_v,
               sem_s, sem_h0, sem_h1):
    wid = lax.axis_index("s") * NC + lax.axis_index("c")

    for f in range(NF):
        pltpu.async_copy(sf_hbm.at[f, pl.ds(wid * BPW, BPW)], sfi_v.at[f],
                         sem_s)
    pltpu.sync_copy(hist_hbm.at[pl.ds(wid * (BPW * L), BPW * L)], hidx_v)

    def _dbl(i, _):
        hidx_v[pl.ds(i * LANES, LANES)] = (
            hidx_v[pl.ds(i * LANES, LANES)] * 2)
        return 0

    lax.fori_loop(0, BPW * L // LANES, _dbl, 0)

    def _hist_copies(e, buf, sem):
        d1 = pltpu.make_async_copy(
            item_hbm.at[hidx_v.at[pl.ds(e * L, 128)]],
            buf.at[pl.ds(0, 128)], sem)
        d2 = pltpu.make_async_copy(
            item_hbm.at[hidx_v.at[pl.ds(e * L + 128, L - 128)]],
            buf.at[pl.ds(128, L - 128)], sem)
        return d1, d2

    def _hstart(e, buf, sem):
        for d in _hist_copies(e, buf, sem):
            d.start()

    def _hwait(e, buf, sem):
        for d in _hist_copies(e, buf, sem):
            d.wait()

    _hstart(0, hbuf0, sem_h0)
    _hstart(1, hbuf1, sem_h1)

    iota = lax.broadcasted_iota(jnp.int32, (LANES,), 0)
    for f in range(NF):
        pltpu.make_async_copy(sf_hbm.at[f, pl.ds(wid * BPW, BPW)],
                              sfi_v.at[f], sem_s).wait()

    def _build_addrs(quarter):
        b0 = quarter * (BPW // 4)

        def _abody(i, _):
            f = i // (BPW // (4 * LANES))
            c = i % (BPW // (4 * LANES))
            v = sfi_v[f, pl.ds(b0 + c * LANES, LANES)]
            w = lax.shift_right_logical(v, 7) * (8 * 128) + (v & 127)
            dst0 = (c * LANES + iota) * (NF * SD) + f * SD
            for d in range(SD):
                src = w + (f * 2 + d // 8) * (782 * 8 * 128) + (d % 8) * 128
                plsc.store_scatter(addr_v, [dst0 + d], src)
            return 0

        lax.fori_loop(0, NF * (BPW // (4 * LANES)), _abody, 0)

    _sgat = lambda: pltpu.make_async_copy(tables_hbm.at[addr_v], svals_v,
                                          sem_s)
    _build_addrs(0)
    _sgat().start()

    def _accum_store(buf, e):
        zero = jnp.zeros((LANES,), jnp.float32)

        def body(i, accs):
            r = i * 8
            a0, a1, a2, a3 = accs
            for rr in range(8):
                a0 = a0 + buf[r + rr, pl.ds(0, LANES)]
                a1 = a1 + buf[r + rr, pl.ds(LANES, LANES)]
                a2 = a2 + buf[r + rr, pl.ds(2 * LANES, LANES)]
                a3 = a3 + buf[r + rr, pl.ds(3 * LANES, LANES)]
            return a0, a1, a2, a3

        a0, a1, a2, a3 = lax.fori_loop(0, L // 8, body, (zero,) * 4)
        acc_v[e, pl.ds(0, LANES)] = a0
        acc_v[e, pl.ds(LANES, LANES)] = a1
        acc_v[e, pl.ds(2 * LANES, LANES)] = a2
        acc_v[e, pl.ds(3 * LANES, LANES)] = a3

    def _outer(g, _):
        e0 = 2 * g
        _hwait(e0, hbuf0, sem_h0)
        _hstart(e0 + 2, hbuf0, sem_h0)
        _accum_store(hbuf0, e0)
        _hwait(e0 + 1, hbuf1, sem_h1)
        _hstart(e0 + 3, hbuf1, sem_h1)
        _accum_store(hbuf1, e0 + 1)
        return 0

    FQ = FPW // 4
    lax.fori_loop(0, 16, _outer, 0)
    for q in range(1, 4):
        _sgat().wait()
        pltpu.sync_copy(svals_v,
                        flat_out.at[pl.ds(wid * FPW + (q - 1) * FQ, FQ)])
        _build_addrs(q)
        _sgat().start()
        lax.fori_loop(16 * q, min(16 * (q + 1), BPW // 2 - 1), _outer, 0)
    _hwait(BPW - 2, hbuf0, sem_h0)
    _accum_store(hbuf0, BPW - 2)
    _hwait(BPW - 1, hbuf1, sem_h1)
    _accum_store(hbuf1, BPW - 1)

    pltpu.sync_copy(acc_v, pooled_out.at[pl.ds(wid * BPW, BPW)])

    _sgat().wait()
    pltpu.sync_copy(svals_v, flat_out.at[pl.ds(wid * FPW + 3 * FQ, FQ)])


def _mlp_body(flat_ref, pooled_ref, hist_ref, w1a_ref, w1b_ref, b1_ref,
              w2_ref, b2_ref, out_ref):
    cnt = jnp.sum((hist_ref[...] != 0).astype(jnp.float32), axis=1,
                  keepdims=True)
    p = pooled_ref[...] / jnp.maximum(cnt, 1.0)
    h = jnp.dot(flat_ref[...], w1a_ref[...],
                preferred_element_type=jnp.float32)
    h = h + jnp.dot(p, w1b_ref[...], preferred_element_type=jnp.float32)
    h = jnp.maximum(h + b1_ref[...], 0.0)
    out_ref[...] = jnp.dot(h, w2_ref[...],
                           preferred_element_type=jnp.float32) + b2_ref[...]


def _mlp(flat, pooled, history, w1a, w1b, b1, w2, b2):
    BT = 1024
    return pl.pallas_call(
        _mlp_body,
        grid=(B // BT,),
        in_specs=[
            pl.BlockSpec((BT, NF * SD), lambda i: (i, 0)),
            pl.BlockSpec((BT, ID), lambda i: (i, 0)),
            pl.BlockSpec((BT, L), lambda i: (i, 0)),
            pl.BlockSpec((NF * SD, DNN), lambda i: (0, 0)),
            pl.BlockSpec((ID, DNN), lambda i: (0, 0)),
            pl.BlockSpec((1, DNN), lambda i: (0, 0)),
            pl.BlockSpec((DNN, H), lambda i: (0, 0)),
            pl.BlockSpec((1, H), lambda i: (0, 0)),
        ],
        out_specs=pl.BlockSpec((BT, H), lambda i: (i, 0)),
        out_shape=jax.ShapeDtypeStruct((B, H), jnp.float32),
    )(flat, pooled, history, w1a, w1b, b1, w2, b2)


def kernel(sparse_feats, history, sparse_tables, item_table, W1, b1, W2, b2):
    sf_t = sparse_feats.T
    hist_flat = history.reshape(B * L)
    SVP = 782 * 128
    tables_flat = (
        jnp.pad(sparse_tables, ((0, 0), (0, SVP - SV), (0, 0)))
        .transpose(0, 2, 1)
        .reshape(NF, 2, 8, 782, 128)
        .transpose(0, 1, 3, 2, 4)
        .reshape(NF * SD * SVP))
    flat, pooled = _sc_gather(sf_t, hist_flat, tables_flat,
                              _item_pack(item_table.T).reshape(2 * IVR, ID))
    return _mlp(flat.reshape(B, NF * SD), pooled, history,
                W1[:NF * SD], W1[NF * SD:], b1.reshape(1, DNN),
                W2, b2.reshape(1, H))
